# baseline (speedup 1.0000x reference)
import functools

import jax
import jax.numpy as jnp
from jax import lax
from jax.experimental import pallas as pl
from jax.experimental.pallas import tpu as pltpu
from jax.experimental.pallas import tpu_sc as plsc

N = 10000
E = 320000
D = 128
H = 128
CLS = 64
G = 64

NC = 2
NS = 16
K = 128
CHE = 157
EPT = CHE * K
EP = NS * EPT
HALF = 5008
AR = 5120
TRASH = 5112
ARS = AR // NS
AZR = 160
PADDST = 1 << 29


@functools.lru_cache(maxsize=None)
def _get_mesh():
    return plsc.VectorSubcoreMesh(core_axis_name="c", subcore_axis_name="s",
                                  num_cores=NC, num_subcores=NS)


def _msg_body(g_hbm, srcs_hbm, dstc_hbm, out_hbm,
              src_v, dst_v, rows0, rows1, acc, sem0, sem1, ssem0, ssem1):
    c = lax.axis_index("c")
    s = lax.axis_index("s")

    def zero(t, _):
        rows0[t // 8, pl.ds((t % 8) * 16, 16)] = jnp.zeros((16,), jnp.float32)
        return 0

    lax.fori_loop(0, K * 8, zero, 0)
    pltpu.sync_copy(rows0, acc.at[pl.ds(s * ARS, K)])
    pltpu.sync_copy(rows0, acc.at[pl.ds(s * ARS + K, K)])
    pltpu.sync_copy(rows0.at[pl.ds(0, ARS - 2 * K)],
                    acc.at[pl.ds(s * ARS + 2 * K, ARS - 2 * K)])

    pltpu.sync_copy(srcs_hbm.at[s], src_v)
    pltpu.sync_copy(dstc_hbm.at[c, s], dst_v)
    plsc.subcore_barrier()

    pltpu.async_copy(g_hbm.at[src_v.at[0]], rows0, sem0)
    pltpu.async_copy(g_hbm.at[src_v.at[1]], rows1, sem1)

    def step(i, _):
        j0 = 2 * i
        pltpu.make_async_copy(g_hbm.at[src_v.at[j0]], rows0, sem0).wait()
        pltpu.async_copy(rows0, acc.at[dst_v.at[j0]], ssem0, add=True)

        pltpu.make_async_copy(g_hbm.at[src_v.at[j0 + 1]], rows1, sem1).wait()
        pltpu.async_copy(rows1, acc.at[dst_v.at[j0 + 1]], ssem1, add=True)

        @pl.when(j0 + 2 < CHE)
        def _():
            pltpu.make_async_copy(rows0, acc.at[dst_v.at[j0]], ssem0).wait()
            pltpu.async_copy(g_hbm.at[src_v.at[j0 + 2]], rows0, sem0)

        @pl.when(j0 + 3 < CHE)
        def _():
            pltpu.make_async_copy(rows1, acc.at[dst_v.at[j0 + 1]], ssem1).wait()
            pltpu.async_copy(g_hbm.at[src_v.at[j0 + 3]], rows1, sem1)

        return 0

    lax.fori_loop(0, CHE // 2, step, 0)
    pltpu.make_async_copy(g_hbm.at[src_v.at[CHE - 1]], rows0, sem0).wait()
    pltpu.async_copy(rows0, acc.at[dst_v.at[CHE - 1]], ssem0, add=True)
    pltpu.make_async_copy(rows0, acc.at[dst_v.at[CHE - 1]], ssem0).wait()
    pltpu.make_async_copy(rows1, acc.at[dst_v.at[CHE - 2]], ssem1).wait()

    plsc.subcore_barrier()
    for i in range(ARS // AZR):
        off = s * ARS + i * AZR
        pltpu.sync_copy(acc.at[pl.ds(off, AZR)], out_hbm.at[c, pl.ds(off, AZR)])


@functools.lru_cache(maxsize=None)
def _msg_kernel():
    return pl.kernel(
        _msg_body,
        out_type=jax.ShapeDtypeStruct((NC, AR, D), jnp.float32),
        mesh=_get_mesh(),
        scratch_types=[
            pltpu.VMEM((CHE, K), jnp.int32),
            pltpu.VMEM((CHE, K), jnp.int32),
            pltpu.VMEM((K, D), jnp.float32),
            pltpu.VMEM((K, D), jnp.float32),
            pltpu.VMEM_SHARED((AR, D), jnp.float32),
            pltpu.SemaphoreType.DMA,
            pltpu.SemaphoreType.DMA,
            pltpu.SemaphoreType.DMA,
            pltpu.SemaphoreType.DMA,
        ],
    )


def _deg_body(dstc_hbm, out_hbm, dst_v, ones_v, zbuf, acc, dsem):
    c = lax.axis_index("c")
    s = lax.axis_index("s")

    def fill(t, _):
        ones_v[t // 8, pl.ds((t % 8) * 16, 16)] = jnp.ones((16,), jnp.float32)
        zbuf[t // 8, pl.ds((t % 8) * 16, 16)] = jnp.zeros((16,), jnp.float32)
        return 0

    lax.fori_loop(0, K * 8, fill, 0)

    for i in range(ARS // K):
        pltpu.sync_copy(zbuf, acc.at[pl.ds(s * ARS + i * K, K)])
    pltpu.sync_copy(zbuf.at[pl.ds(0, ARS - (ARS // K) * K)],
                    acc.at[pl.ds(s * ARS + (ARS // K) * K,
                                 ARS - (ARS // K) * K)])
    pltpu.sync_copy(dstc_hbm.at[c, s], dst_v)
    plsc.subcore_barrier()

    def count(j, _):
        pltpu.async_copy(ones_v, acc.at[dst_v.at[j]], dsem, add=True)
        return 0

    lax.fori_loop(0, CHE, count, 0)

    def drain(j, _):
        pltpu.make_async_copy(ones_v, acc.at[dst_v.at[j]], dsem).wait()
        return 0

    lax.fori_loop(0, CHE, drain, 0)

    plsc.subcore_barrier()
    for i in range(ARS // AZR):
        off = s * ARS + i * AZR
        pltpu.sync_copy(acc.at[pl.ds(off, AZR)], out_hbm.at[c, pl.ds(off, AZR)])


@functools.lru_cache(maxsize=None)
def _deg_kernel():
    return pl.kernel(
        _deg_body,
        out_type=jax.ShapeDtypeStruct((NC, AR, D), jnp.float32),
        mesh=_get_mesh(),
        scratch_types=[
            pltpu.VMEM((CHE, K), jnp.int32),
            pltpu.VMEM((K, D), jnp.float32),
            pltpu.VMEM((K, D), jnp.float32),
            pltpu.VMEM_SHARED((AR, D), jnp.float32),
            pltpu.SemaphoreType.DMA,
        ],
    )


def _dst_split_body(dst_ref, out_ref):
    d = dst_ref[...]
    for c in range(NC):
        local = d - c * HALF
        ok = (local >= 0) & (local < HALF)
        out_ref[c] = jnp.where(ok, local, TRASH)


def _tc_first_body(degp_ref, x_ref, w_ref, g_ref, dis_ref):
    deg = jnp.concatenate([degp_ref[0, :HALF, 0:1],
                           degp_ref[1, : N - HALF, 0:1]], axis=0) + 1.0
    dis = lax.rsqrt(jnp.maximum(deg, 1.0))
    dis_ref[...] = dis
    g_ref[...] = jnp.dot(x_ref[...], w_ref[...],
                         preferred_element_type=jnp.float32) * dis


def _gcn_combine(p_ref, g_ref, dis_ref, b_ref):
    scat = jnp.concatenate([p_ref[0, :HALF], p_ref[1, : N - HALF]], axis=0)
    return dis_ref[...] * (scat + g_ref[...]) + b_ref[...]


def _tc_mid_body(p_ref, g_ref, dis_ref, w_ref, b_ref, gout_ref):
    h = _gcn_combine(p_ref, g_ref, dis_ref, b_ref)
    gout_ref[...] = jnp.dot(h, w_ref[...],
                            preferred_element_type=jnp.float32) * dis_ref[...]


def _tc_final_body(p_ref, g_ref, dis_ref, b_ref, batch_ref,
                   l1w_ref, l1b_ref, l2w_ref, l2b_ref, fcw_ref, fcb_ref,
                   out_ref):
    h = _gcn_combine(p_ref, g_ref, dis_ref, b_ref)
    gids = lax.broadcasted_iota(jnp.int32, (1, G), 1)
    onehot = (batch_ref[...] == gids).astype(jnp.float32)
    dn = (((0,), (0,)), ((), ()))
    sums = lax.dot_general(onehot, h, dn,
                           preferred_element_type=jnp.float32)
    counts = lax.dot_general(onehot, jnp.ones((N, 1), jnp.float32), dn,
                             preferred_element_type=jnp.float32)
    pooled = sums / jnp.maximum(counts, 1.0)
    r = jnp.dot(pooled, l1w_ref[...],
                preferred_element_type=jnp.float32) + l1b_ref[...]
    r = jnp.dot(r, l2w_ref[...],
                preferred_element_type=jnp.float32) + l2b_ref[...]
    r = jnp.dot(r, fcw_ref[...],
                preferred_element_type=jnp.float32) + fcb_ref[...]
    out_ref[...] = jnp.where(r >= 0, r, r * 0.01)


_dst_split = pl.pallas_call(
    _dst_split_body,
    out_shape=jax.ShapeDtypeStruct((NC, EP // 128, 128), jnp.int32),
)

_tc_first = pl.pallas_call(
    _tc_first_body,
    out_shape=[jax.ShapeDtypeStruct((N, D), jnp.float32),
               jax.ShapeDtypeStruct((N, 1), jnp.float32)],
)

_tc_mid = pl.pallas_call(
    _tc_mid_body,
    out_shape=jax.ShapeDtypeStruct((N, D), jnp.float32),
)

_tc_final = pl.pallas_call(
    _tc_final_body,
    out_shape=jax.ShapeDtypeStruct((G, 1), jnp.float32),
)


def kernel(x, edge_index, batch, W0, b0, W1, b1, W2, b2,
           l1W, l1b, l2W, l2b, fcW, fcb):
    pad = EP - E
    srcp = jnp.concatenate(
        [edge_index[0], jnp.zeros((pad,), edge_index.dtype)])
    dstp = jnp.concatenate(
        [edge_index[1], jnp.full((pad,), PADDST, edge_index.dtype)])
    srcs = srcp.reshape(NS, CHE, K)
    dstc = _dst_split(dstp.reshape(EP // 128, 128)).reshape(NC, NS, CHE, K)

    deg_p = _deg_kernel()(dstc)
    g1, dis = _tc_first(deg_p, x, W0)

    msg = _msg_kernel()
    p1 = msg(g1, srcs, dstc)
    g2 = _tc_mid(p1, g1, dis, W1, b0.reshape(1, H))

    p2 = msg(g2, srcs, dstc)
    g3 = _tc_mid(p2, g2, dis, W2, b1.reshape(1, H))

    p3 = msg(g3, srcs, dstc)
    return _tc_final(p3, g3, dis, b2.reshape(1, H), batch.reshape(N, 1),
                     l1W, l1b.reshape(1, CLS), l2W, l2b.reshape(1, 1),
                     fcW, fcb.reshape(1, 1))


# speedup vs baseline: 1.0801x; 1.0801x over previous
import functools

import jax
import jax.numpy as jnp
from jax import lax
from jax.experimental import pallas as pl
from jax.experimental.pallas import tpu as pltpu
from jax.experimental.pallas import tpu_sc as plsc

N = 10000
E = 320000
D = 128
H = 128
CLS = 64
G = 64

NC = 2
NS = 16
K = 128
CHE = 157
EPT = CHE * K
EP = NS * EPT
HALF = 5008
AR = 5120
TRASH = 5112
ARS = AR // NS
AZR = 160
PADDST = 1 << 29


@functools.lru_cache(maxsize=None)
def _get_mesh():
    return plsc.VectorSubcoreMesh(core_axis_name="c", subcore_axis_name="s",
                                  num_cores=NC, num_subcores=NS)


def _msg_body(g_hbm, srcs_hbm, dstc_hbm, out_hbm,
              src_v, dst_v, rows0, rows1, acc, sem0, sem1, ssem0, ssem1):
    c = lax.axis_index("c")
    s = lax.axis_index("s")

    def zero(t, _):
        rows0[t // 8, pl.ds((t % 8) * 16, 16)] = jnp.zeros((16,), jnp.float32)
        return 0

    lax.fori_loop(0, K * 8, zero, 0)
    pltpu.sync_copy(rows0, acc.at[pl.ds(s * ARS, K)])
    pltpu.sync_copy(rows0, acc.at[pl.ds(s * ARS + K, K)])
    pltpu.sync_copy(rows0.at[pl.ds(0, ARS - 2 * K)],
                    acc.at[pl.ds(s * ARS + 2 * K, ARS - 2 * K)])

    pltpu.sync_copy(srcs_hbm.at[s], src_v)
    pltpu.sync_copy(dstc_hbm.at[c, s], dst_v)
    plsc.subcore_barrier()

    pltpu.async_copy(g_hbm.at[src_v.at[0]], rows0, sem0)
    pltpu.async_copy(g_hbm.at[src_v.at[1]], rows1, sem1)

    def step(i, _):
        j0 = 2 * i
        pltpu.make_async_copy(g_hbm.at[src_v.at[j0]], rows0, sem0).wait()
        pltpu.sync_copy(rows0, acc.at[dst_v.at[j0]], add=True)

        @pl.when(j0 + 2 < CHE)
        def _():
            pltpu.async_copy(g_hbm.at[src_v.at[j0 + 2]], rows0, sem0)

        pltpu.make_async_copy(g_hbm.at[src_v.at[j0 + 1]], rows1, sem1).wait()
        pltpu.sync_copy(rows1, acc.at[dst_v.at[j0 + 1]], add=True)

        @pl.when(j0 + 3 < CHE)
        def _():
            pltpu.async_copy(g_hbm.at[src_v.at[j0 + 3]], rows1, sem1)

        return 0

    lax.fori_loop(0, CHE // 2, step, 0)
    pltpu.make_async_copy(g_hbm.at[src_v.at[CHE - 1]], rows0, sem0).wait()
    pltpu.sync_copy(rows0, acc.at[dst_v.at[CHE - 1]], add=True)

    plsc.subcore_barrier()
    for i in range(ARS // AZR):
        off = s * ARS + i * AZR
        pltpu.sync_copy(acc.at[pl.ds(off, AZR)], out_hbm.at[c, pl.ds(off, AZR)])


@functools.lru_cache(maxsize=None)
def _msg_kernel():
    return pl.kernel(
        _msg_body,
        out_type=jax.ShapeDtypeStruct((NC, AR, D), jnp.float32),
        mesh=_get_mesh(),
        scratch_types=[
            pltpu.VMEM((CHE, K), jnp.int32),
            pltpu.VMEM((CHE, K), jnp.int32),
            pltpu.VMEM((K, D), jnp.float32),
            pltpu.VMEM((K, D), jnp.float32),
            pltpu.VMEM_SHARED((AR, D), jnp.float32),
            pltpu.SemaphoreType.DMA,
            pltpu.SemaphoreType.DMA,
            pltpu.SemaphoreType.DMA,
            pltpu.SemaphoreType.DMA,
        ],
    )


def _deg_body(dstc_hbm, out_hbm, dst_v, ones_v, zbuf, acc, dsem):
    c = lax.axis_index("c")
    s = lax.axis_index("s")

    def fill(t, _):
        ones_v[t // 8, pl.ds((t % 8) * 16, 16)] = jnp.ones((16,), jnp.float32)
        zbuf[t // 8, pl.ds((t % 8) * 16, 16)] = jnp.zeros((16,), jnp.float32)
        return 0

    lax.fori_loop(0, K * 8, fill, 0)

    for i in range(ARS // K):
        pltpu.sync_copy(zbuf, acc.at[pl.ds(s * ARS + i * K, K)])
    pltpu.sync_copy(zbuf.at[pl.ds(0, ARS - (ARS // K) * K)],
                    acc.at[pl.ds(s * ARS + (ARS // K) * K,
                                 ARS - (ARS // K) * K)])
    pltpu.sync_copy(dstc_hbm.at[c, s], dst_v)
    plsc.subcore_barrier()

    def count(j, _):
        pltpu.async_copy(ones_v, acc.at[dst_v.at[j]], dsem, add=True)
        return 0

    lax.fori_loop(0, CHE, count, 0)

    def drain(j, _):
        pltpu.make_async_copy(ones_v, acc.at[dst_v.at[j]], dsem).wait()
        return 0

    lax.fori_loop(0, CHE, drain, 0)

    plsc.subcore_barrier()
    for i in range(ARS // AZR):
        off = s * ARS + i * AZR
        pltpu.sync_copy(acc.at[pl.ds(off, AZR)], out_hbm.at[c, pl.ds(off, AZR)])


@functools.lru_cache(maxsize=None)
def _deg_kernel():
    return pl.kernel(
        _deg_body,
        out_type=jax.ShapeDtypeStruct((NC, AR, D), jnp.float32),
        mesh=_get_mesh(),
        scratch_types=[
            pltpu.VMEM((CHE, K), jnp.int32),
            pltpu.VMEM((K, D), jnp.float32),
            pltpu.VMEM((K, D), jnp.float32),
            pltpu.VMEM_SHARED((AR, D), jnp.float32),
            pltpu.SemaphoreType.DMA,
        ],
    )


def _dst_split_body(dst_ref, out_ref):
    d = dst_ref[...]
    for c in range(NC):
        local = d - c * HALF
        ok = (local >= 0) & (local < HALF)
        out_ref[c] = jnp.where(ok, local, TRASH)


def _tc_first_body(degp_ref, x_ref, w_ref, g_ref, dis_ref):
    deg = jnp.concatenate([degp_ref[0, :HALF, 0:1],
                           degp_ref[1, : N - HALF, 0:1]], axis=0) + 1.0
    dis = lax.rsqrt(jnp.maximum(deg, 1.0))
    dis_ref[...] = dis
    g_ref[...] = jnp.dot(x_ref[...], w_ref[...],
                         preferred_element_type=jnp.float32) * dis


def _gcn_combine(p_ref, g_ref, dis_ref, b_ref):
    scat = jnp.concatenate([p_ref[0, :HALF], p_ref[1, : N - HALF]], axis=0)
    return dis_ref[...] * (scat + g_ref[...]) + b_ref[...]


def _tc_mid_body(p_ref, g_ref, dis_ref, w_ref, b_ref, gout_ref):
    h = _gcn_combine(p_ref, g_ref, dis_ref, b_ref)
    gout_ref[...] = jnp.dot(h, w_ref[...],
                            preferred_element_type=jnp.float32) * dis_ref[...]


def _tc_final_body(p_ref, g_ref, dis_ref, b_ref, batch_ref,
                   l1w_ref, l1b_ref, l2w_ref, l2b_ref, fcw_ref, fcb_ref,
                   out_ref):
    h = _gcn_combine(p_ref, g_ref, dis_ref, b_ref)
    gids = lax.broadcasted_iota(jnp.int32, (1, G), 1)
    onehot = (batch_ref[...] == gids).astype(jnp.float32)
    dn = (((0,), (0,)), ((), ()))
    sums = lax.dot_general(onehot, h, dn,
                           preferred_element_type=jnp.float32)
    counts = lax.dot_general(onehot, jnp.ones((N, 1), jnp.float32), dn,
                             preferred_element_type=jnp.float32)
    pooled = sums / jnp.maximum(counts, 1.0)
    r = jnp.dot(pooled, l1w_ref[...],
                preferred_element_type=jnp.float32) + l1b_ref[...]
    r = jnp.dot(r, l2w_ref[...],
                preferred_element_type=jnp.float32) + l2b_ref[...]
    r = jnp.dot(r, fcw_ref[...],
                preferred_element_type=jnp.float32) + fcb_ref[...]
    out_ref[...] = jnp.where(r >= 0, r, r * 0.01)


_dst_split = pl.pallas_call(
    _dst_split_body,
    out_shape=jax.ShapeDtypeStruct((NC, EP // 128, 128), jnp.int32),
)

_tc_first = pl.pallas_call(
    _tc_first_body,
    out_shape=[jax.ShapeDtypeStruct((N, D), jnp.float32),
               jax.ShapeDtypeStruct((N, 1), jnp.float32)],
)

_tc_mid = pl.pallas_call(
    _tc_mid_body,
    out_shape=jax.ShapeDtypeStruct((N, D), jnp.float32),
)

_tc_final = pl.pallas_call(
    _tc_final_body,
    out_shape=jax.ShapeDtypeStruct((G, 1), jnp.float32),
)


def kernel(x, edge_index, batch, W0, b0, W1, b1, W2, b2,
           l1W, l1b, l2W, l2b, fcW, fcb):
    pad = EP - E
    srcp = jnp.concatenate(
        [edge_index[0], jnp.zeros((pad,), edge_index.dtype)])
    dstp = jnp.concatenate(
        [edge_index[1], jnp.full((pad,), PADDST, edge_index.dtype)])
    srcs = srcp.reshape(NS, CHE, K)
    dstc = _dst_split(dstp.reshape(EP // 128, 128)).reshape(NC, NS, CHE, K)

    deg_p = _deg_kernel()(dstc)
    g1, dis = _tc_first(deg_p, x, W0)

    msg = _msg_kernel()
    p1 = msg(g1, srcs, dstc)
    g2 = _tc_mid(p1, g1, dis, W1, b0.reshape(1, H))

    p2 = msg(g2, srcs, dstc)
    g3 = _tc_mid(p2, g2, dis, W2, b1.reshape(1, H))

    p3 = msg(g3, srcs, dstc)
    return _tc_final(p3, g3, dis, b2.reshape(1, H), batch.reshape(N, 1),
                     l1W, l1b.reshape(1, CLS), l2W, l2b.reshape(1, 1),
                     fcW, fcb.reshape(1, 1))


# speedup vs baseline: 1.6881x; 1.5630x over previous
import functools

import jax
import jax.numpy as jnp
from jax import lax
from jax.experimental import pallas as pl
from jax.experimental.pallas import tpu as pltpu
from jax.experimental.pallas import tpu_sc as plsc

N = 10000
E = 320000
D = 128
H = 128
CLS = 64
G = 64

NC = 2
NS = 16
K = 128
CHE = 157
EPT = CHE * K
EP = NS * EPT
HALF = 5008
AR = 5120
TRASH = 5112
ARS = AR // NS
AZR = 160
PADDST = 1 << 29
PCH = 160
SS = 272


@functools.lru_cache(maxsize=None)
def _get_mesh():
    return plsc.VectorSubcoreMesh(core_axis_name="c", subcore_axis_name="s",
                                  num_cores=NC, num_subcores=NS)


def _part_body(srcs_hbm, dstc_hbm, psrc_hbm, pdst_hbm, cnt_hbm,
               in_s, in_d, st_s, st_d, out_s, out_d, cnt_v):
    c = lax.axis_index("c")
    s = lax.axis_index("s")
    pltpu.sync_copy(srcs_hbm.at[s], in_s)
    pltpu.sync_copy(dstc_hbm.at[c, s], in_d)

    trash = jnp.full((16,), TRASH, jnp.int32)
    zeros = jnp.zeros((16,), jnp.int32)

    def prefill(t, _):
        st_s[pl.ds(t * 16, 16)] = zeros
        st_d[pl.ds(t * 16, 16)] = trash
        return 0

    lax.fori_loop(0, SS // 16, prefill, 0)

    def flush_row(r_out):
        for q in range(K // 16):
            out_s[r_out, pl.ds(q * 16, 16)] = st_s[pl.ds(q * 16, 16)]
            out_d[r_out, pl.ds(q * 16, 16)] = st_d[pl.ds(q * 16, 16)]

    lanes = lax.iota(jnp.int32, 16)

    def vec(t, carry):
        n_local, r_out = carry
        r = t // 8
        l = t % 8
        dv = in_d[r, pl.ds(l * 16, 16)]
        sv = in_s[r, pl.ds(l * 16, 16)]
        mask = dv < TRASH
        keys = lanes + jnp.where(mask, 0, 1024)
        _, sdst = plsc.sort_key_val(keys, dv)
        _, ssrc = plsc.sort_key_val(keys, sv)
        st_d[pl.ds(n_local, 16)] = sdst
        st_s[pl.ds(n_local, 16)] = ssrc
        n_local = n_local + jnp.sum(mask.astype(jnp.int32))
        do_flush = n_local >= K

        @pl.when(do_flush)
        def _():
            flush_row(r_out)
            for q in range((SS - K) // 16):
                st_s[pl.ds(q * 16, 16)] = st_s[pl.ds(K + q * 16, 16)]
                st_d[pl.ds(q * 16, 16)] = st_d[pl.ds(K + q * 16, 16)]
            for q in range(K // 16):
                st_d[pl.ds(SS - K + q * 16, 16)] = trash

        n_local = jnp.where(do_flush, n_local - K, n_local)
        r_out = jnp.where(do_flush, r_out + 1, r_out)
        return (n_local, r_out)

    n_local, r_out = lax.fori_loop(0, CHE * 8, vec, (0, 0))

    flush_row(r_out)
    def retrash(t, _):
        st_s[pl.ds(t * 16, 16)] = zeros
        st_d[pl.ds(t * 16, 16)] = trash
        return 0

    lax.fori_loop(0, K // 16, retrash, 0)
    flush_row(r_out + 1)

    nch2 = ((r_out + 2) // 2) * 2
    cvec = jnp.broadcast_to(nch2, (16,)).astype(jnp.int32)

    def putc(t, _):
        cnt_v[0, pl.ds(t * 16, 16)] = cvec
        return 0

    lax.fori_loop(0, K // 16, putc, 0)

    pltpu.sync_copy(out_s, psrc_hbm.at[c, s])
    pltpu.sync_copy(out_d, pdst_hbm.at[c, s])
    pltpu.sync_copy(cnt_v, cnt_hbm.at[c, s])


@functools.lru_cache(maxsize=None)
def _part_kernel():
    return pl.kernel(
        _part_body,
        out_type=[jax.ShapeDtypeStruct((NC, NS, PCH, K), jnp.int32),
                  jax.ShapeDtypeStruct((NC, NS, PCH, K), jnp.int32),
                  jax.ShapeDtypeStruct((NC, NS, 1, K), jnp.int32)],
        mesh=_get_mesh(),
        scratch_types=[
            pltpu.VMEM((CHE, K), jnp.int32),
            pltpu.VMEM((CHE, K), jnp.int32),
            pltpu.VMEM((SS,), jnp.int32),
            pltpu.VMEM((SS,), jnp.int32),
            pltpu.VMEM((PCH, K), jnp.int32),
            pltpu.VMEM((PCH, K), jnp.int32),
            pltpu.VMEM((1, K), jnp.int32),
        ],
        compiler_params=pltpu.CompilerParams(needs_layout_passes=False),
    )


def _msg_body(g_hbm, psrc_hbm, pdst_hbm, cnt_hbm, out_hbm,
              src_v, dst_v, cnt_v, rows0, rows1, acc, sem0, sem1):
    c = lax.axis_index("c")
    s = lax.axis_index("s")

    def zero(t, _):
        rows0[t // 8, pl.ds((t % 8) * 16, 16)] = jnp.zeros((16,), jnp.float32)
        return 0

    lax.fori_loop(0, K * 8, zero, 0)
    pltpu.sync_copy(rows0, acc.at[pl.ds(s * ARS, K)])
    pltpu.sync_copy(rows0, acc.at[pl.ds(s * ARS + K, K)])
    pltpu.sync_copy(rows0.at[pl.ds(0, ARS - 2 * K)],
                    acc.at[pl.ds(s * ARS + 2 * K, ARS - 2 * K)])

    pltpu.sync_copy(psrc_hbm.at[c, s], src_v)
    pltpu.sync_copy(pdst_hbm.at[c, s], dst_v)
    pltpu.sync_copy(cnt_hbm.at[c, s], cnt_v)
    nch2 = jnp.max(cnt_v[0, pl.ds(0, 16)], axis=0)
    plsc.subcore_barrier()

    pltpu.async_copy(g_hbm.at[src_v.at[0]], rows0, sem0)
    pltpu.async_copy(g_hbm.at[src_v.at[1]], rows1, sem1)

    def step(i, _):
        j0 = 2 * i
        pltpu.make_async_copy(g_hbm.at[src_v.at[j0]], rows0, sem0).wait()
        pltpu.sync_copy(rows0, acc.at[dst_v.at[j0]], add=True)

        @pl.when(j0 + 2 < nch2)
        def _():
            pltpu.async_copy(g_hbm.at[src_v.at[j0 + 2]], rows0, sem0)

        pltpu.make_async_copy(g_hbm.at[src_v.at[j0 + 1]], rows1, sem1).wait()
        pltpu.sync_copy(rows1, acc.at[dst_v.at[j0 + 1]], add=True)

        @pl.when(j0 + 3 < nch2)
        def _():
            pltpu.async_copy(g_hbm.at[src_v.at[j0 + 3]], rows1, sem1)

        return 0

    lax.fori_loop(0, nch2 // 2, step, 0)

    plsc.subcore_barrier()
    for i in range(ARS // AZR):
        off = s * ARS + i * AZR
        pltpu.sync_copy(acc.at[pl.ds(off, AZR)], out_hbm.at[c, pl.ds(off, AZR)])


@functools.lru_cache(maxsize=None)
def _msg_kernel():
    return pl.kernel(
        _msg_body,
        out_type=jax.ShapeDtypeStruct((NC, AR, D), jnp.float32),
        mesh=_get_mesh(),
        scratch_types=[
            pltpu.VMEM((PCH, K), jnp.int32),
            pltpu.VMEM((PCH, K), jnp.int32),
            pltpu.VMEM((1, K), jnp.int32),
            pltpu.VMEM((K, D), jnp.float32),
            pltpu.VMEM((K, D), jnp.float32),
            pltpu.VMEM_SHARED((AR, D), jnp.float32),
            pltpu.SemaphoreType.DMA,
            pltpu.SemaphoreType.DMA,
        ],
        compiler_params=pltpu.CompilerParams(needs_layout_passes=False),
    )


def _deg_body(pdst_hbm, cnt_hbm, out_hbm, dst_v, cnt_v, ones_v, zbuf, acc,
              dsem):
    c = lax.axis_index("c")
    s = lax.axis_index("s")

    def fill(t, _):
        ones_v[t // 8, pl.ds((t % 8) * 16, 16)] = jnp.ones((16,), jnp.float32)
        zbuf[t // 8, pl.ds((t % 8) * 16, 16)] = jnp.zeros((16,), jnp.float32)
        return 0

    lax.fori_loop(0, K * 8, fill, 0)

    for i in range(ARS // K):
        pltpu.sync_copy(zbuf, acc.at[pl.ds(s * ARS + i * K, K)])
    pltpu.sync_copy(zbuf.at[pl.ds(0, ARS - (ARS // K) * K)],
                    acc.at[pl.ds(s * ARS + (ARS // K) * K,
                                 ARS - (ARS // K) * K)])
    pltpu.sync_copy(pdst_hbm.at[c, s], dst_v)
    pltpu.sync_copy(cnt_hbm.at[c, s], cnt_v)
    nch2 = jnp.max(cnt_v[0, pl.ds(0, 16)], axis=0)
    plsc.subcore_barrier()

    def count(j, _):
        pltpu.async_copy(ones_v, acc.at[dst_v.at[j]], dsem, add=True)
        return 0

    lax.fori_loop(0, nch2, count, 0)

    def drain(j, _):
        pltpu.make_async_copy(ones_v, acc.at[dst_v.at[j]], dsem).wait()
        return 0

    lax.fori_loop(0, nch2, drain, 0)

    plsc.subcore_barrier()
    for i in range(ARS // AZR):
        off = s * ARS + i * AZR
        pltpu.sync_copy(acc.at[pl.ds(off, AZR)], out_hbm.at[c, pl.ds(off, AZR)])


@functools.lru_cache(maxsize=None)
def _deg_kernel():
    return pl.kernel(
        _deg_body,
        out_type=jax.ShapeDtypeStruct((NC, AR, D), jnp.float32),
        mesh=_get_mesh(),
        scratch_types=[
            pltpu.VMEM((PCH, K), jnp.int32),
            pltpu.VMEM((1, K), jnp.int32),
            pltpu.VMEM((K, D), jnp.float32),
            pltpu.VMEM((K, D), jnp.float32),
            pltpu.VMEM_SHARED((AR, D), jnp.float32),
            pltpu.SemaphoreType.DMA,
        ],
        compiler_params=pltpu.CompilerParams(needs_layout_passes=False),
    )


def _dst_split_body(dst_ref, out_ref):
    d = dst_ref[...]
    for c in range(NC):
        local = d - c * HALF
        ok = (local >= 0) & (local < HALF)
        out_ref[c] = jnp.where(ok, local, TRASH)


def _tc_first_body(degp_ref, x_ref, w_ref, g_ref, dis_ref):
    deg = jnp.concatenate([degp_ref[0, :HALF, 0:1],
                           degp_ref[1, : N - HALF, 0:1]], axis=0) + 1.0
    dis = lax.rsqrt(jnp.maximum(deg, 1.0))
    dis_ref[...] = dis
    g_ref[...] = jnp.dot(x_ref[...], w_ref[...],
                         preferred_element_type=jnp.float32) * dis


def _gcn_combine(p_ref, g_ref, dis_ref, b_ref):
    scat = jnp.concatenate([p_ref[0, :HALF], p_ref[1, : N - HALF]], axis=0)
    return dis_ref[...] * (scat + g_ref[...]) + b_ref[...]


def _tc_mid_body(p_ref, g_ref, dis_ref, w_ref, b_ref, gout_ref):
    h = _gcn_combine(p_ref, g_ref, dis_ref, b_ref)
    gout_ref[...] = jnp.dot(h, w_ref[...],
                            preferred_element_type=jnp.float32) * dis_ref[...]


def _tc_final_body(p_ref, g_ref, dis_ref, b_ref, batch_ref,
                   l1w_ref, l1b_ref, l2w_ref, l2b_ref, fcw_ref, fcb_ref,
                   out_ref):
    h = _gcn_combine(p_ref, g_ref, dis_ref, b_ref)
    gids = lax.broadcasted_iota(jnp.int32, (1, G), 1)
    onehot = (batch_ref[...] == gids).astype(jnp.float32)
    dn = (((0,), (0,)), ((), ()))
    sums = lax.dot_general(onehot, h, dn,
                           preferred_element_type=jnp.float32)
    counts = lax.dot_general(onehot, jnp.ones((N, 1), jnp.float32), dn,
                             preferred_element_type=jnp.float32)
    pooled = sums / jnp.maximum(counts, 1.0)
    r = jnp.dot(pooled, l1w_ref[...],
                preferred_element_type=jnp.float32) + l1b_ref[...]
    r = jnp.dot(r, l2w_ref[...],
                preferred_element_type=jnp.float32) + l2b_ref[...]
    r = jnp.dot(r, fcw_ref[...],
                preferred_element_type=jnp.float32) + fcb_ref[...]
    out_ref[...] = jnp.where(r >= 0, r, r * 0.01)


_dst_split = pl.pallas_call(
    _dst_split_body,
    out_shape=jax.ShapeDtypeStruct((NC, EP // 128, 128), jnp.int32),
)

_tc_first = pl.pallas_call(
    _tc_first_body,
    out_shape=[jax.ShapeDtypeStruct((N, D), jnp.float32),
               jax.ShapeDtypeStruct((N, 1), jnp.float32)],
)

_tc_mid = pl.pallas_call(
    _tc_mid_body,
    out_shape=jax.ShapeDtypeStruct((N, D), jnp.float32),
)

_tc_final = pl.pallas_call(
    _tc_final_body,
    out_shape=jax.ShapeDtypeStruct((G, 1), jnp.float32),
)


def kernel(x, edge_index, batch, W0, b0, W1, b1, W2, b2,
           l1W, l1b, l2W, l2b, fcW, fcb):
    pad = EP - E
    srcp = jnp.concatenate(
        [edge_index[0], jnp.zeros((pad,), edge_index.dtype)])
    dstp = jnp.concatenate(
        [edge_index[1], jnp.full((pad,), PADDST, edge_index.dtype)])
    srcs = srcp.reshape(NS, CHE, K)
    dstc = _dst_split(dstp.reshape(EP // 128, 128)).reshape(NC, NS, CHE, K)

    psrc, pdst, cnt = _part_kernel()(srcs, dstc)
    deg_p = _deg_kernel()(pdst, cnt)
    g1, dis = _tc_first(deg_p, x, W0)

    msg = _msg_kernel()
    p1 = msg(g1, psrc, pdst, cnt)
    g2 = _tc_mid(p1, g1, dis, W1, b0.reshape(1, H))

    p2 = msg(g2, psrc, pdst, cnt)
    g3 = _tc_mid(p2, g2, dis, W2, b1.reshape(1, H))

    p3 = msg(g3, psrc, pdst, cnt)
    return _tc_final(p3, g3, dis, b2.reshape(1, H), batch.reshape(N, 1),
                     l1W, l1b.reshape(1, CLS), l2W, l2b.reshape(1, 1),
                     fcW, fcb.reshape(1, 1))
